# baseline (speedup 1.0000x reference)
import functools

import jax
import jax.numpy as jnp
from jax import lax
from jax.experimental import pallas as pl
from jax.experimental.pallas import tpu as pltpu
from jax.experimental.pallas import tpu_sc as plsc

L = 16
NC = 2
NS = 16
NW = NC * NS
F = 26
GC = 128
BLK = 128

_DN = lax.GatherDimensionNumbers(
    offset_dims=(), collapsed_slice_dims=(0,), start_index_map=(0,))


def _perm(vec, idx):
    return lax.gather(vec, idx[:, None], _DN, (1,),
                      mode=lax.GatherScatterMode.PROMISE_IN_BOUNDS)


def _bcast_lane(vec, lane):
    return _perm(vec, jnp.full((L,), lane, dtype=jnp.int32))


def _lanesum(x, iota):
    for sh in (8, 4, 2, 1):
        x = x + _perm(x, jnp.bitwise_xor(iota, sh))
    return x


@functools.lru_cache(maxsize=None)
def _build(B):
    RPW = B // NW
    CH = 32
    CHN = CH * F
    NCHUNK = RPW // CH
    slices = []
    o = 0
    while o < CHN:
        n = min(GC, CHN - o)
        slices.append((o, n))
        o += n

    mesh = plsc.VectorSubcoreMesh(core_axis_name="c", subcore_axis_name="s")

    @functools.partial(
        pl.kernel,
        out_type=jax.ShapeDtypeStruct((B,), jnp.float32),
        mesh=mesh,
        compiler_params=pltpu.CompilerParams(use_tc_tiling_on_sc=True),
        scratch_types=[
            pltpu.VMEM((CHN + L,), jnp.int32),
            pltpu.VMEM((CHN,), jnp.int32),
            pltpu.VMEM((CHN + L,), jnp.float32),
            pltpu.VMEM((CHN, BLK), jnp.float32),
            pltpu.VMEM((CHN + L,), jnp.float32),
            pltpu.VMEM((CH,), jnp.float32),
            pltpu.VMEM((L,), jnp.float32),
            pltpu.SemaphoreType.DMA,
            pltpu.SemaphoreType.DMA,
        ],
    )
    def fm(idx_hbm, idx8_hbm, vals_hbm, w_hbm, Vb_hbm, b_hbm, y_hbm,
           idxv, idx8v, vvals, vblk, wrows, ybuf, bv, sem_v, sem_w):
        cid = lax.axis_index("c")
        sid = lax.axis_index("s")
        wid = sid * NC + cid
        base = wid * RPW
        pltpu.sync_copy(b_hbm, bv)
        iota = lax.iota(jnp.int32, L)
        m10 = iota < (F - L)
        fzero = jnp.zeros((L,), jnp.float32)

        def chunk(ci, carry):
            rowbase = base + ci * CH
            pltpu.sync_copy(idx_hbm.at[pl.ds(rowbase * F, CHN)],
                            idxv.at[pl.ds(0, CHN)])
            pltpu.sync_copy(idx8_hbm.at[pl.ds(rowbase * F, CHN)], idx8v)
            pltpu.sync_copy(vals_hbm.at[pl.ds(rowbase * F, CHN)],
                            vvals.at[pl.ds(0, CHN)])
            cps = []
            for (so, sn) in slices:
                sl = pl.ds(so, sn)
                cps.append(pltpu.async_copy(
                    Vb_hbm.at[idx8v.at[sl]], vblk.at[sl, :], sem_v))
                cps.append(pltpu.async_copy(
                    w_hbm.at[idxv.at[sl]], wrows.at[sl], sem_w))
            for cp in cps:
                cp.wait()

            bvec = bv[...]

            def grp(g, c2):
                def row_body(rr, lvec):
                    r = g * L + rr
                    off = r * F
                    va = vvals[pl.ds(off, L)]
                    vb = vvals[pl.ds(off + L, L)]
                    o8a = (idxv[pl.ds(off, L)] & 7) * L
                    o8b = (idxv[pl.ds(off + L, L)] & 7) * L
                    accxv = fzero
                    accx2 = fzero
                    for f in range(F):
                        if f < L:
                            bf = _bcast_lane(va, f)
                            o8 = o8a[f]
                        else:
                            bf = _bcast_lane(vb, f - L)
                            o8 = o8b[f - L]
                        j = off + f
                        t = bf * vblk[j, pl.ds(o8, L)]
                        accxv = accxv + t
                        accx2 = accx2 + t * t
                    d = accxv * accxv - accx2
                    wa = wrows[pl.ds(off, L)]
                    wb = wrows[pl.ds(off + L, L)]
                    vbm = jnp.where(m10, vb, 0.0)
                    s = _lanesum(d, iota)
                    sumv = _lanesum(va + vbm, iota)
                    xw = _lanesum(va * wa + jnp.where(m10, vb * wb, 0.0),
                                  iota)
                    logit = xw + 0.5 * s / sumv
                    return jnp.where(iota == rr, logit, lvec)

                lvec = lax.fori_loop(0, L, row_body, fzero)
                y = 1.0 / (1.0 + jnp.exp(-(lvec + bvec)))
                ybuf[pl.ds(g * L, L)] = y
                return c2

            lax.fori_loop(0, CH // L, grp, 0)
            pltpu.sync_copy(ybuf, y_hbm.at[pl.ds(rowbase, CH)])
            return carry

        lax.fori_loop(0, NCHUNK, chunk, 0)

    return fm


def kernel(indices, values, w, V, b):
    B = indices.shape[0]
    idx_flat = indices.reshape(-1).astype(jnp.int32)
    idx8_flat = jnp.right_shift(idx_flat, 3)
    vals_flat = values.reshape(-1).astype(jnp.float32)
    w_flat = w.reshape(-1).astype(jnp.float32)
    Vb = jnp.reshape(V.astype(jnp.float32), (V.shape[0] * L // BLK, BLK))
    b16 = jnp.zeros((L,), jnp.float32) + b.reshape(-1)[0].astype(jnp.float32)
    return _build(B)(idx_flat, idx8_flat, vals_flat, w_flat, Vb, b16)


# speedup vs baseline: 1.0437x; 1.0437x over previous
import functools

import jax
import jax.numpy as jnp
from jax import lax
from jax.experimental import pallas as pl
from jax.experimental.pallas import tpu as pltpu
from jax.experimental.pallas import tpu_sc as plsc

L = 16
NC = 2
NS = 16
NW = NC * NS
F = 26
GC = 128
BLK = 128

_DN = lax.GatherDimensionNumbers(
    offset_dims=(), collapsed_slice_dims=(0,), start_index_map=(0,))


def _perm(vec, idx):
    return lax.gather(vec, idx[:, None], _DN, (1,),
                      mode=lax.GatherScatterMode.PROMISE_IN_BOUNDS)


def _bcast_lane(vec, lane):
    return _perm(vec, jnp.full((L,), lane, dtype=jnp.int32))


def _lanesum(x, iota):
    for sh in (8, 4, 2, 1):
        x = x + _perm(x, jnp.bitwise_xor(iota, sh))
    return x


def _tc_relayout(Vt):
    NV = Vt.shape[1]
    NB = NV // 8
    C = 1024
    grid = NB // C

    def body(*refs):
        y_ref = refs[8]
        y_ref[...] = jnp.concatenate(
            [jnp.transpose(refs[t][...]) for t in range(8)], axis=1)

    return pl.pallas_call(
        body,
        grid=(grid,),
        in_specs=[
            pl.BlockSpec((L, C), lambda i, t=t: (0, t * (NB // C) + i))
            for t in range(8)
        ],
        out_specs=pl.BlockSpec((C, BLK), lambda i: (i, 0)),
        out_shape=jax.ShapeDtypeStruct((NB, BLK), jnp.float32),
        compiler_params=pltpu.CompilerParams(
            dimension_semantics=("arbitrary",)),
    )(*([Vt] * 8))


@functools.lru_cache(maxsize=None)
def _build(B):
    RPW = B // NW
    CH = 32
    CHN = CH * F
    NCHUNK = RPW // CH
    slices = []
    o = 0
    while o < CHN:
        n = min(GC, CHN - o)
        slices.append((o, n))
        o += n

    mesh = plsc.VectorSubcoreMesh(core_axis_name="c", subcore_axis_name="s")

    @functools.partial(
        pl.kernel,
        out_type=jax.ShapeDtypeStruct((B,), jnp.float32),
        mesh=mesh,
        compiler_params=pltpu.CompilerParams(use_tc_tiling_on_sc=True),
        scratch_types=[
            pltpu.VMEM((CHN + L,), jnp.int32),
            pltpu.VMEM((CHN,), jnp.int32),
            pltpu.VMEM((CHN + L,), jnp.int32),
            pltpu.VMEM((CHN + L,), jnp.float32),
            pltpu.VMEM((CHN, BLK), jnp.float32),
            pltpu.VMEM((CHN + L,), jnp.float32),
            pltpu.VMEM((CH,), jnp.float32),
            pltpu.VMEM((L,), jnp.float32),
            pltpu.SemaphoreType.DMA,
            pltpu.SemaphoreType.DMA,
        ],
    )
    def fm(idx_hbm, idx8_hbm, odiv_hbm, vals_hbm, w_hbm, Vb_hbm, b_hbm,
           y_hbm, idxv, idx8v, odivv, vvals, vblk, wrows, ybuf, bv,
           sem_v, sem_w):
        cid = lax.axis_index("c")
        sid = lax.axis_index("s")
        wid = sid * NC + cid
        base = wid * RPW
        pltpu.sync_copy(b_hbm, bv)
        iota = lax.iota(jnp.int32, L)
        m10 = iota < (F - L)
        fzero = jnp.zeros((L,), jnp.float32)

        def chunk(ci, carry):
            rowbase = base + ci * CH
            pltpu.sync_copy(idx_hbm.at[pl.ds(rowbase * F, CHN)],
                            idxv.at[pl.ds(0, CHN)])
            pltpu.sync_copy(idx8_hbm.at[pl.ds(rowbase * F, CHN)], idx8v)
            pltpu.sync_copy(odiv_hbm.at[pl.ds(rowbase * F, CHN)],
                            odivv.at[pl.ds(0, CHN)])
            pltpu.sync_copy(vals_hbm.at[pl.ds(rowbase * F, CHN)],
                            vvals.at[pl.ds(0, CHN)])
            cps = []
            for (so, sn) in slices:
                sl = pl.ds(so, sn)
                cps.append(pltpu.async_copy(
                    Vb_hbm.at[idx8v.at[sl]], vblk.at[sl, :], sem_v))
                cps.append(pltpu.async_copy(
                    w_hbm.at[idxv.at[sl]], wrows.at[sl], sem_w))
            for cp in cps:
                cp.wait()

            bvec = bv[...]

            def grp(g, c2):
                def row_body(rr, lvec):
                    r = g * L + rr
                    off = r * F
                    va = vvals[pl.ds(off, L)]
                    vb = vvals[pl.ds(off + L, L)]
                    o8a = odivv[pl.ds(off, L)]
                    o8b = odivv[pl.ds(off + L, L)]
                    accxv = fzero
                    accx2 = fzero
                    for f in range(F):
                        if f < L:
                            bf = _bcast_lane(va, f)
                            o8 = o8a[f]
                        else:
                            bf = _bcast_lane(vb, f - L)
                            o8 = o8b[f - L]
                        j = off + f
                        t = bf * vblk[j, pl.ds(o8, L)]
                        accxv = accxv + t
                        accx2 = accx2 + t * t
                    d = accxv * accxv - accx2
                    wa = wrows[pl.ds(off, L)]
                    wb = wrows[pl.ds(off + L, L)]
                    vbm = jnp.where(m10, vb, 0.0)
                    s = _lanesum(d, iota)
                    sumv = _lanesum(va + vbm, iota)
                    xw = _lanesum(va * wa + jnp.where(m10, vb * wb, 0.0),
                                  iota)
                    logit = xw + 0.5 * s / sumv
                    return jnp.where(iota == rr, logit, lvec)

                lvec = lax.fori_loop(0, L, row_body, fzero)
                y = 1.0 / (1.0 + jnp.exp(-(lvec + bvec)))
                ybuf[pl.ds(g * L, L)] = y
                return c2

            lax.fori_loop(0, CH // L, grp, 0)
            pltpu.sync_copy(ybuf, y_hbm.at[pl.ds(rowbase, CH)])
            return carry

        lax.fori_loop(0, NCHUNK, chunk, 0)

    return fm


def kernel(indices, values, w, V, b):
    B = indices.shape[0]
    NB = 128000
    idx_flat = indices.reshape(-1).astype(jnp.int32)
    idxm_flat = idx_flat % NB
    odiv_flat = (idx_flat // NB) * L
    vals_flat = values.reshape(-1).astype(jnp.float32)
    w_flat = w.reshape(-1).astype(jnp.float32)
    Vt = V.astype(jnp.float32).T
    Vtp = jnp.pad(Vt, ((0, 0), (0, NB * 8 - Vt.shape[1])))
    Vb = _tc_relayout(Vtp)
    b16 = jnp.zeros((L,), jnp.float32) + b.reshape(-1)[0].astype(jnp.float32)
    return _build(B)(idx_flat, idxm_flat, odiv_flat, vals_flat, w_flat,
                     Vb, b16)


# speedup vs baseline: 1.5124x; 1.4491x over previous
import functools

import jax
import jax.numpy as jnp
from jax import lax
from jax.experimental import pallas as pl
from jax.experimental.pallas import tpu as pltpu
from jax.experimental.pallas import tpu_sc as plsc

L = 16
NC = 2
NS = 16
NW = NC * NS
F = 26
GC = 128
BLK = 128

_DN = lax.GatherDimensionNumbers(
    offset_dims=(), collapsed_slice_dims=(0,), start_index_map=(0,))


def _perm(vec, idx):
    return lax.gather(vec, idx[:, None], _DN, (1,),
                      mode=lax.GatherScatterMode.PROMISE_IN_BOUNDS)


def _bcast_lane(vec, lane):
    return _perm(vec, jnp.full((L,), lane, dtype=jnp.int32))


def _lanesum(x, iota):
    for sh in (8, 4, 2, 1):
        x = x + _perm(x, jnp.bitwise_xor(iota, sh))
    return x


def _tc_relayout(Vt):
    NV = Vt.shape[1]
    NB = NV // 8
    C = 1024
    grid = NB // C

    def body(*refs):
        y_ref = refs[8]
        y_ref[...] = jnp.transpose(jnp.concatenate(
            [refs[t][...] for t in range(8)], axis=0))

    return pl.pallas_call(
        body,
        grid=(grid,),
        in_specs=[
            pl.BlockSpec((L, C), lambda i, t=t: (0, t * (NB // C) + i))
            for t in range(8)
        ],
        out_specs=pl.BlockSpec((C, BLK), lambda i: (i, 0)),
        out_shape=jax.ShapeDtypeStruct((NB, BLK), jnp.float32),
        compiler_params=pltpu.CompilerParams(
            dimension_semantics=("arbitrary",)),
    )(*([Vt] * 8))


@functools.lru_cache(maxsize=None)
def _build(B):
    RPW = B // NW
    CH = 32
    CHN = CH * F
    NCHUNK = RPW // CH
    slices = []
    o = 0
    while o < CHN:
        n = min(GC, CHN - o)
        slices.append((o, n))
        o += n

    mesh = plsc.VectorSubcoreMesh(core_axis_name="c", subcore_axis_name="s")

    @functools.partial(
        pl.kernel,
        out_type=jax.ShapeDtypeStruct((B,), jnp.float32),
        mesh=mesh,
        compiler_params=pltpu.CompilerParams(use_tc_tiling_on_sc=True),
        scratch_types=[
            pltpu.VMEM((CHN + L,), jnp.int32),
            pltpu.VMEM((CHN,), jnp.int32),
            pltpu.VMEM((CHN + L,), jnp.int32),
            pltpu.VMEM((CHN + L,), jnp.float32),
            pltpu.VMEM((CHN, BLK), jnp.float32),
            pltpu.VMEM((CHN + L,), jnp.float32),
            pltpu.VMEM((CH,), jnp.float32),
            pltpu.VMEM((L,), jnp.float32),
            pltpu.SemaphoreType.DMA,
            pltpu.SemaphoreType.DMA,
        ],
    )
    def fm(idx_hbm, idx8_hbm, odiv_hbm, vals_hbm, w_hbm, Vb_hbm, b_hbm,
           y_hbm, idxv, idx8v, odivv, vvals, vblk, wrows, ybuf, bv,
           sem_v, sem_w):
        cid = lax.axis_index("c")
        sid = lax.axis_index("s")
        wid = sid * NC + cid
        base = wid * RPW
        pltpu.sync_copy(b_hbm, bv)
        iota = lax.iota(jnp.int32, L)
        m10 = iota < (F - L)
        fzero = jnp.zeros((L,), jnp.float32)

        def chunk(ci, carry):
            rowbase = base + ci * CH
            pltpu.sync_copy(idx_hbm.at[pl.ds(rowbase * F, CHN)],
                            idxv.at[pl.ds(0, CHN)])
            pltpu.sync_copy(idx8_hbm.at[pl.ds(rowbase * F, CHN)], idx8v)
            pltpu.sync_copy(odiv_hbm.at[pl.ds(rowbase * F, CHN)],
                            odivv.at[pl.ds(0, CHN)])
            pltpu.sync_copy(vals_hbm.at[pl.ds(rowbase * F, CHN)],
                            vvals.at[pl.ds(0, CHN)])
            cps = []
            for (so, sn) in slices:
                sl = pl.ds(so, sn)
                cps.append(pltpu.async_copy(
                    Vb_hbm.at[idx8v.at[sl]], vblk.at[sl, :], sem_v))
                cps.append(pltpu.async_copy(
                    w_hbm.at[idxv.at[sl]], wrows.at[sl], sem_w))
            for cp in cps:
                cp.wait()

            bvec = bv[...]

            def grp(g, c2):
                def row_body(rr, lvec):
                    r = g * L + rr
                    off = r * F
                    va = vvals[pl.ds(off, L)]
                    vb = vvals[pl.ds(off + L, L)]
                    o8a = odivv[pl.ds(off, L)]
                    o8b = odivv[pl.ds(off + L, L)]
                    accxv = fzero
                    accx2 = fzero
                    for f in range(F):
                        if f < L:
                            bf = _bcast_lane(va, f)
                            o8 = o8a[f]
                        else:
                            bf = _bcast_lane(vb, f - L)
                            o8 = o8b[f - L]
                        j = off + f
                        t = bf * vblk[j, pl.ds(o8, L)]
                        accxv = accxv + t
                        accx2 = accx2 + t * t
                    d = accxv * accxv - accx2
                    wa = wrows[pl.ds(off, L)]
                    wb = wrows[pl.ds(off + L, L)]
                    vbm = jnp.where(m10, vb, 0.0)
                    s = _lanesum(d, iota)
                    sumv = _lanesum(va + vbm, iota)
                    xw = _lanesum(va * wa + jnp.where(m10, vb * wb, 0.0),
                                  iota)
                    logit = xw + 0.5 * s / sumv
                    return jnp.where(iota == rr, logit, lvec)

                lvec = lax.fori_loop(0, L, row_body, fzero)
                y = 1.0 / (1.0 + jnp.exp(-(lvec + bvec)))
                ybuf[pl.ds(g * L, L)] = y
                return c2

            lax.fori_loop(0, CH // L, grp, 0)
            pltpu.sync_copy(ybuf, y_hbm.at[pl.ds(rowbase, CH)])
            return carry

        lax.fori_loop(0, NCHUNK, chunk, 0)

    return fm


def kernel(indices, values, w, V, b):
    B = indices.shape[0]
    NB = 128000
    idx_flat = indices.reshape(-1).astype(jnp.int32)
    idxm_flat = idx_flat % NB
    odiv_flat = (idx_flat // NB) * L
    vals_flat = values.reshape(-1).astype(jnp.float32)
    w_flat = w.astype(jnp.float32).T.reshape(-1)
    Vt = V.astype(jnp.float32).T
    Vtp = jnp.pad(Vt, ((0, 0), (0, NB * 8 - Vt.shape[1])))
    Vb = _tc_relayout(Vtp)
    b16 = jnp.zeros((L,), jnp.float32) + b.reshape(-1)[0].astype(jnp.float32)
    return _build(B)(idx_flat, idxm_flat, odiv_flat, vals_flat, w_flat,
                     Vb, b16)


# speedup vs baseline: 1.9607x; 1.2964x over previous
import functools

import jax
import jax.numpy as jnp
from jax import lax
from jax.experimental import pallas as pl
from jax.experimental.pallas import tpu as pltpu
from jax.experimental.pallas import tpu_sc as plsc

L = 16
NC = 2
NS = 16
NW = NC * NS
F = 26
GC = 128
BLK = 128

_DN = lax.GatherDimensionNumbers(
    offset_dims=(), collapsed_slice_dims=(0,), start_index_map=(0,))


def _perm(vec, idx):
    return lax.gather(vec, idx[:, None], _DN, (1,),
                      mode=lax.GatherScatterMode.PROMISE_IN_BOUNDS)


def _bcast_lane(vec, lane):
    return _perm(vec, jnp.full((L,), lane, dtype=jnp.int32))


def _lanesum(x, iota):
    for sh in (8, 4, 2, 1):
        x = x + _perm(x, jnp.bitwise_xor(iota, sh))
    return x


def _tc_relayout(Vt):
    NV = Vt.shape[1]
    NB = NV // 8
    C = 1024
    grid = NB // C

    def body(*refs):
        y_ref = refs[8]
        y_ref[...] = jnp.transpose(jnp.concatenate(
            [refs[t][...] for t in range(8)], axis=0))

    return pl.pallas_call(
        body,
        grid=(grid,),
        in_specs=[
            pl.BlockSpec((L, C), lambda i, t=t: (0, t * (NB // C) + i))
            for t in range(8)
        ],
        out_specs=pl.BlockSpec((C, BLK), lambda i: (i, 0)),
        out_shape=jax.ShapeDtypeStruct((NB, BLK), jnp.float32),
        compiler_params=pltpu.CompilerParams(
            dimension_semantics=("arbitrary",)),
    )(*([Vt] * 8))


@functools.lru_cache(maxsize=None)
def _build(B):
    RPW = B // NW
    CH = 32
    CHN = CH * F
    NCHUNK = RPW // CH
    slices = []
    o = 0
    while o < CHN:
        n = min(GC, CHN - o)
        slices.append((o, n))
        o += n

    mesh = plsc.VectorSubcoreMesh(core_axis_name="c", subcore_axis_name="s")

    @functools.partial(
        pl.kernel,
        out_type=jax.ShapeDtypeStruct((B,), jnp.float32),
        mesh=mesh,
        compiler_params=pltpu.CompilerParams(use_tc_tiling_on_sc=False),
        scratch_types=[
            pltpu.VMEM((CHN + L,), jnp.int32),
            pltpu.VMEM((CHN,), jnp.int32),
            pltpu.VMEM((CHN + L,), jnp.float32),
            pltpu.VMEM((CHN, L), jnp.float32),
            pltpu.VMEM((CHN + L,), jnp.float32),
            pltpu.VMEM((CH,), jnp.float32),
            pltpu.VMEM((L,), jnp.float32),
            pltpu.SemaphoreType.DMA,
            pltpu.SemaphoreType.DMA,
        ],
    )
    def fm(idx_hbm, idx8_hbm, vals_hbm, w_hbm, Vb_hbm, b_hbm,
           y_hbm, idxv, idx8v, vvals, vrows, wrows, ybuf, bv,
           sem_v, sem_w):
        cid = lax.axis_index("c")
        sid = lax.axis_index("s")
        wid = sid * NC + cid
        base = wid * RPW
        pltpu.sync_copy(b_hbm, bv)
        iota = lax.iota(jnp.int32, L)
        m10 = iota < (F - L)
        fzero = jnp.zeros((L,), jnp.float32)

        def chunk(ci, carry):
            rowbase = base + ci * CH
            pltpu.sync_copy(idx_hbm.at[pl.ds(rowbase * F, CHN)],
                            idxv.at[pl.ds(0, CHN)])
            pltpu.sync_copy(idx8_hbm.at[pl.ds(rowbase * F, CHN)], idx8v)
            pltpu.sync_copy(vals_hbm.at[pl.ds(rowbase * F, CHN)],
                            vvals.at[pl.ds(0, CHN)])
            cps = []
            for (so, sn) in slices:
                sl = pl.ds(so, sn)
                cps.append(pltpu.async_copy(
                    Vb_hbm.at[idx8v.at[sl]], vrows.at[sl, :], sem_v))
                cps.append(pltpu.async_copy(
                    w_hbm.at[idxv.at[sl]], wrows.at[sl], sem_w))
            for cp in cps:
                cp.wait()

            bvec = bv[...]

            def grp(g, c2):
                def row_body(rr, lvec):
                    r = g * L + rr
                    off = r * F
                    va = vvals[pl.ds(off, L)]
                    vb = vvals[pl.ds(off + L, L)]
                    accxv = fzero
                    accx2 = fzero
                    for f in range(F):
                        if f < L:
                            bf = _bcast_lane(va, f)
                        else:
                            bf = _bcast_lane(vb, f - L)
                        j = off + f
                        t = bf * vrows[j, :]
                        accxv = accxv + t
                        accx2 = accx2 + t * t
                    d = accxv * accxv - accx2
                    wa = wrows[pl.ds(off, L)]
                    wb = wrows[pl.ds(off + L, L)]
                    vbm = jnp.where(m10, vb, 0.0)
                    s = _lanesum(d, iota)
                    sumv = _lanesum(va + vbm, iota)
                    xw = _lanesum(va * wa + jnp.where(m10, vb * wb, 0.0),
                                  iota)
                    logit = xw + 0.5 * s / sumv
                    return jnp.where(iota == rr, logit, lvec)

                lvec = lax.fori_loop(0, L, row_body, fzero)
                y = 1.0 / (1.0 + jnp.exp(-(lvec + bvec)))
                ybuf[pl.ds(g * L, L)] = y
                return c2

            lax.fori_loop(0, CH // L, grp, 0)
            pltpu.sync_copy(ybuf, y_hbm.at[pl.ds(rowbase, CH)])
            return carry

        lax.fori_loop(0, NCHUNK, chunk, 0)

    return fm


def kernel(indices, values, w, V, b):
    B = indices.shape[0]
    NB = 128000
    idx_flat = indices.reshape(-1).astype(jnp.int32)
    idxp_flat = (idx_flat % NB) * 8 + idx_flat // NB
    vals_flat = values.reshape(-1).astype(jnp.float32)
    w_flat = w.astype(jnp.float32).T.reshape(-1)
    Vt = V.astype(jnp.float32).T
    Vtp = jnp.pad(Vt, ((0, 0), (0, NB * 8 - Vt.shape[1])))
    Vlin = _tc_relayout(Vtp).reshape(NB * 8, L)
    b16 = jnp.zeros((L,), jnp.float32) + b.reshape(-1)[0].astype(jnp.float32)
    return _build(B)(idx_flat, idxp_flat, vals_flat, w_flat,
                     Vlin, b16)


# speedup vs baseline: 3.3092x; 1.6878x over previous
import functools

import jax
import jax.numpy as jnp
from jax import lax
from jax.experimental import pallas as pl
from jax.experimental.pallas import tpu as pltpu
from jax.experimental.pallas import tpu_sc as plsc

L = 16
NC = 2
NS = 16
NW = NC * NS
F = 26
GC = 128
BLK = 128

_DN = lax.GatherDimensionNumbers(
    offset_dims=(), collapsed_slice_dims=(0,), start_index_map=(0,))


def _perm(vec, idx):
    return lax.gather(vec, idx[:, None], _DN, (1,),
                      mode=lax.GatherScatterMode.PROMISE_IN_BOUNDS)


def _bcast_lane(vec, lane):
    return _perm(vec, jnp.full((L,), lane, dtype=jnp.int32))


def _lanesum(x, iota):
    for sh in (8, 4, 2, 1):
        x = x + _perm(x, jnp.bitwise_xor(iota, sh))
    return x


def _tc_relayout(Vt, pad7, NB):
    C = 6400
    grid = NB // C

    def body(*refs):
        y_ref = refs[8]
        y_ref[...] = jnp.transpose(jnp.concatenate(
            [refs[t][...] for t in range(8)], axis=0))

    return pl.pallas_call(
        body,
        grid=(grid,),
        in_specs=[
            pl.BlockSpec((L, C), lambda i, t=t: (0, t * (NB // C) + i))
            for t in range(7)
        ] + [pl.BlockSpec((L, C), lambda i: (0, i))],
        out_specs=pl.BlockSpec((C, BLK), lambda i: (i, 0)),
        out_shape=jax.ShapeDtypeStruct((NB, BLK), jnp.float32),
        compiler_params=pltpu.CompilerParams(
            dimension_semantics=("arbitrary",)),
    )(*([Vt] * 7 + [pad7]))


def _tc_wflat(wt):
    NWD = wt.shape[1]

    def body(x_ref, y_ref):
        y_ref[...] = x_ref[0, :]

    return pl.pallas_call(
        body,
        out_shape=jax.ShapeDtypeStruct((NWD,), jnp.float32),
    )(wt)


@functools.lru_cache(maxsize=None)
def _build(B):
    RPW = B // NW
    CH = 32
    CHN = CH * F
    NCHUNK = RPW // CH
    slices = []
    o = 0
    while o < CHN:
        n = min(GC, CHN - o)
        slices.append((o, n))
        o += n

    mesh = plsc.VectorSubcoreMesh(core_axis_name="c", subcore_axis_name="s")

    @functools.partial(
        pl.kernel,
        out_type=jax.ShapeDtypeStruct((B,), jnp.float32),
        mesh=mesh,
        compiler_params=pltpu.CompilerParams(use_tc_tiling_on_sc=False),
        scratch_types=[
            pltpu.VMEM((CHN + L,), jnp.int32),
            pltpu.VMEM((CHN,), jnp.int32),
            pltpu.VMEM((CHN + L,), jnp.float32),
            pltpu.VMEM((CHN, L), jnp.float32),
            pltpu.VMEM((CHN + L,), jnp.float32),
            pltpu.VMEM((CH,), jnp.float32),
            pltpu.VMEM((L,), jnp.float32),
            pltpu.SemaphoreType.DMA,
            pltpu.SemaphoreType.DMA,
        ],
    )
    def fm(idx_hbm, idx8_hbm, vals_hbm, w_hbm, Vb_hbm, b_hbm,
           y_hbm, idxv, idx8v, vvals, vrows, wrows, ybuf, bv,
           sem_v, sem_w):
        cid = lax.axis_index("c")
        sid = lax.axis_index("s")
        wid = sid * NC + cid
        base = wid * RPW
        pltpu.sync_copy(b_hbm, bv)
        iota = lax.iota(jnp.int32, L)
        m10 = iota < (F - L)
        fzero = jnp.zeros((L,), jnp.float32)

        def chunk(ci, carry):
            rowbase = base + ci * CH
            pltpu.sync_copy(idx_hbm.at[pl.ds(rowbase * F, CHN)],
                            idxv.at[pl.ds(0, CHN)])
            pltpu.sync_copy(idx8_hbm.at[pl.ds(rowbase * F, CHN)], idx8v)
            pltpu.sync_copy(vals_hbm.at[pl.ds(rowbase * F, CHN)],
                            vvals.at[pl.ds(0, CHN)])
            cps = []
            for (so, sn) in slices:
                sl = pl.ds(so, sn)
                cps.append(pltpu.async_copy(
                    Vb_hbm.at[idx8v.at[sl]], vrows.at[sl, :], sem_v))
                cps.append(pltpu.async_copy(
                    w_hbm.at[idxv.at[sl]], wrows.at[sl], sem_w))
            for cp in cps:
                cp.wait()

            bvec = bv[...]

            def grp(g, c2):
                def row_body(rr, lvec):
                    r = g * L + rr
                    off = r * F
                    va = vvals[pl.ds(off, L)]
                    vb = vvals[pl.ds(off + L, L)]
                    accxv = fzero
                    accx2 = fzero
                    for f in range(F):
                        if f < L:
                            bf = _bcast_lane(va, f)
                        else:
                            bf = _bcast_lane(vb, f - L)
                        j = off + f
                        t = bf * vrows[j, :]
                        accxv = accxv + t
                        accx2 = accx2 + t * t
                    d = accxv * accxv - accx2
                    wa = wrows[pl.ds(off, L)]
                    wb = wrows[pl.ds(off + L, L)]
                    vbm = jnp.where(m10, vb, 0.0)
                    s = _lanesum(d, iota)
                    sumv = _lanesum(va + vbm, iota)
                    xw = _lanesum(va * wa + jnp.where(m10, vb * wb, 0.0),
                                  iota)
                    logit = xw + 0.5 * s / sumv
                    return jnp.where(iota == rr, logit, lvec)

                lvec = lax.fori_loop(0, L, row_body, fzero)
                y = 1.0 / (1.0 + jnp.exp(-(lvec + bvec)))
                ybuf[pl.ds(g * L, L)] = y
                return c2

            lax.fori_loop(0, CH // L, grp, 0)
            pltpu.sync_copy(ybuf, y_hbm.at[pl.ds(rowbase, CH)])
            return carry

        lax.fori_loop(0, NCHUNK, chunk, 0)

    return fm


def kernel(indices, values, w, V, b):
    B = indices.shape[0]
    NB = 128000
    idx_flat = indices.reshape(-1).astype(jnp.int32)
    idxp_flat = (idx_flat % NB) * 8 + idx_flat // NB
    vals_flat = values.reshape(-1).astype(jnp.float32)
    w_flat = _tc_wflat(w.astype(jnp.float32).T)
    Vt = V.astype(jnp.float32).T
    NV = Vt.shape[1]
    pad7 = jnp.pad(Vt[:, 7 * NB:], ((0, 0), (0, NB * 8 - NV)))
    Vlin = _tc_relayout(Vt, pad7, NB).reshape(NB * 8, L)
    b16 = jnp.zeros((L,), jnp.float32) + b.reshape(-1)[0].astype(jnp.float32)
    return _build(B)(idx_flat, idxp_flat, vals_flat, w_flat,
                     Vlin, b16)


# speedup vs baseline: 3.7156x; 1.1228x over previous
import functools

import jax
import jax.numpy as jnp
from jax import lax
from jax.experimental import pallas as pl
from jax.experimental.pallas import tpu as pltpu
from jax.experimental.pallas import tpu_sc as plsc

L = 16
NC = 2
NS = 16
NW = NC * NS
F = 26
GC = 128
BLK = 128

_DN = lax.GatherDimensionNumbers(
    offset_dims=(), collapsed_slice_dims=(0,), start_index_map=(0,))


def _perm(vec, idx):
    return lax.gather(vec, idx[:, None], _DN, (1,),
                      mode=lax.GatherScatterMode.PROMISE_IN_BOUNDS)


def _bcast_lane(vec, lane):
    return _perm(vec, jnp.full((L,), lane, dtype=jnp.int32))


def _lanesum(x, iota):
    for sh in (8, 4, 2, 1):
        x = x + _perm(x, jnp.bitwise_xor(iota, sh))
    return x


def _tc_relayout(Vt, pad7, NB):
    C = 6400
    grid = NB // C

    def body(*refs):
        y_ref = refs[8]
        y_ref[...] = jnp.transpose(jnp.concatenate(
            [refs[t][...] for t in range(8)], axis=0))

    return pl.pallas_call(
        body,
        grid=(grid,),
        in_specs=[
            pl.BlockSpec((L, C), lambda i, t=t: (0, t * (NB // C) + i))
            for t in range(7)
        ] + [pl.BlockSpec((L, C), lambda i: (0, i))],
        out_specs=pl.BlockSpec((C, BLK), lambda i: (i, 0)),
        out_shape=jax.ShapeDtypeStruct((NB, BLK), jnp.float32),
        compiler_params=pltpu.CompilerParams(
            dimension_semantics=("arbitrary",)),
    )(*([Vt] * 7 + [pad7]))


def _tc_wflat(wt):
    NWD = wt.shape[1]

    def body(x_ref, y_ref):
        y_ref[...] = x_ref[0, :]

    return pl.pallas_call(
        body,
        out_shape=jax.ShapeDtypeStruct((NWD,), jnp.float32),
    )(wt)


@functools.lru_cache(maxsize=None)
def _build(B):
    RPW = B // NW
    CH = 128
    CHN = CH * F
    NCHUNK = RPW // CH
    slices = []
    o = 0
    while o < CHN:
        n = min(GC, CHN - o)
        slices.append((o, n))
        o += n

    mesh = plsc.VectorSubcoreMesh(core_axis_name="c", subcore_axis_name="s")

    @functools.partial(
        pl.kernel,
        out_type=jax.ShapeDtypeStruct((B,), jnp.float32),
        mesh=mesh,
        compiler_params=pltpu.CompilerParams(use_tc_tiling_on_sc=False),
        scratch_types=[
            pltpu.VMEM((CHN + L,), jnp.int32),
            pltpu.VMEM((CHN,), jnp.int32),
            pltpu.VMEM((CHN + L,), jnp.float32),
            pltpu.VMEM((CHN, L), jnp.float32),
            pltpu.VMEM((CHN + L,), jnp.float32),
            pltpu.VMEM((CH,), jnp.float32),
            pltpu.VMEM((L,), jnp.float32),
            pltpu.SemaphoreType.DMA,
            pltpu.SemaphoreType.DMA,
        ],
    )
    def fm(idx_hbm, idx8_hbm, vals_hbm, w_hbm, Vb_hbm, b_hbm,
           y_hbm, idxv, idx8v, vvals, vrows, wrows, ybuf, bv,
           sem_v, sem_w):
        cid = lax.axis_index("c")
        sid = lax.axis_index("s")
        wid = sid * NC + cid
        base = wid * RPW
        pltpu.sync_copy(b_hbm, bv)
        iota = lax.iota(jnp.int32, L)
        m10 = iota < (F - L)
        fzero = jnp.zeros((L,), jnp.float32)

        def chunk(ci, carry):
            rowbase = base + ci * CH
            pltpu.sync_copy(idx_hbm.at[pl.ds(rowbase * F, CHN)],
                            idxv.at[pl.ds(0, CHN)])
            pltpu.sync_copy(idx8_hbm.at[pl.ds(rowbase * F, CHN)], idx8v)
            pltpu.sync_copy(vals_hbm.at[pl.ds(rowbase * F, CHN)],
                            vvals.at[pl.ds(0, CHN)])
            cps = []
            for (so, sn) in slices:
                sl = pl.ds(so, sn)
                cps.append(pltpu.async_copy(
                    Vb_hbm.at[idx8v.at[sl]], vrows.at[sl, :], sem_v))
                cps.append(pltpu.async_copy(
                    w_hbm.at[idxv.at[sl]], wrows.at[sl], sem_w))
            for cp in cps:
                cp.wait()

            bvec = bv[...]

            def grp(g, c2):
                def row_body(rr, lvec):
                    r = g * L + rr
                    off = r * F
                    va = vvals[pl.ds(off, L)]
                    vb = vvals[pl.ds(off + L, L)]
                    accxv = fzero
                    accx2 = fzero
                    for f in range(F):
                        if f < L:
                            bf = _bcast_lane(va, f)
                        else:
                            bf = _bcast_lane(vb, f - L)
                        j = off + f
                        t = bf * vrows[j, :]
                        accxv = accxv + t
                        accx2 = accx2 + t * t
                    d = accxv * accxv - accx2
                    wa = wrows[pl.ds(off, L)]
                    wb = wrows[pl.ds(off + L, L)]
                    vbm = jnp.where(m10, vb, 0.0)
                    s = _lanesum(d, iota)
                    sumv = _lanesum(va + vbm, iota)
                    xw = _lanesum(va * wa + jnp.where(m10, vb * wb, 0.0),
                                  iota)
                    logit = xw + 0.5 * s / sumv
                    return jnp.where(iota == rr, logit, lvec)

                lvec = lax.fori_loop(0, L, row_body, fzero)
                y = 1.0 / (1.0 + jnp.exp(-(lvec + bvec)))
                ybuf[pl.ds(g * L, L)] = y
                return c2

            lax.fori_loop(0, CH // L, grp, 0)
            pltpu.sync_copy(ybuf, y_hbm.at[pl.ds(rowbase, CH)])
            return carry

        lax.fori_loop(0, NCHUNK, chunk, 0)

    return fm


def kernel(indices, values, w, V, b):
    B = indices.shape[0]
    NB = 128000
    idx_flat = indices.reshape(-1).astype(jnp.int32)
    idxp_flat = (idx_flat % NB) * 8 + idx_flat // NB
    vals_flat = values.reshape(-1).astype(jnp.float32)
    w_flat = _tc_wflat(w.astype(jnp.float32).T)
    Vt = V.astype(jnp.float32).T
    NV = Vt.shape[1]
    pad7 = jnp.pad(Vt[:, 7 * NB:], ((0, 0), (0, NB * 8 - NV)))
    Vlin = _tc_relayout(Vt, pad7, NB).reshape(NB * 8, L)
    b16 = jnp.zeros((L,), jnp.float32) + b.reshape(-1)[0].astype(jnp.float32)
    return _build(B)(idx_flat, idxp_flat, vals_flat, w_flat,
                     Vlin, b16)


# speedup vs baseline: 4.0352x; 1.0860x over previous
import functools

import jax
import jax.numpy as jnp
from jax import lax
from jax.experimental import pallas as pl
from jax.experimental.pallas import tpu as pltpu
from jax.experimental.pallas import tpu_sc as plsc

L = 16
NC = 2
NS = 16
NW = NC * NS
F = 26
GC = 128
BLK = 128

_DN = lax.GatherDimensionNumbers(
    offset_dims=(), collapsed_slice_dims=(0,), start_index_map=(0,))


def _perm(vec, idx):
    return lax.gather(vec, idx[:, None], _DN, (1,),
                      mode=lax.GatherScatterMode.PROMISE_IN_BOUNDS)


def _bcast_lane(vec, lane):
    return _perm(vec, jnp.full((L,), lane, dtype=jnp.int32))


def _lanesum(x, iota):
    for sh in (8, 4, 2, 1):
        x = x + _perm(x, jnp.bitwise_xor(iota, sh))
    return x


def _tc_relayout(Vt, pad7, NB):
    C = 6400
    grid = NB // C

    def body(*refs):
        y_ref = refs[8]
        y_ref[...] = jnp.transpose(jnp.concatenate(
            [refs[t][...] for t in range(8)], axis=0))

    return pl.pallas_call(
        body,
        grid=(grid,),
        in_specs=[
            pl.BlockSpec((L, C), lambda i, t=t: (0, t * (NB // C) + i))
            for t in range(7)
        ] + [pl.BlockSpec((L, C), lambda i: (0, i))],
        out_specs=pl.BlockSpec((C, BLK), lambda i: (i, 0)),
        out_shape=jax.ShapeDtypeStruct((NB, BLK), jnp.float32),
        compiler_params=pltpu.CompilerParams(
            dimension_semantics=("arbitrary",)),
    )(*([Vt] * 7 + [pad7]))


def _tc_wflat(wt):
    NWD = wt.shape[1]

    def body(x_ref, y_ref):
        y_ref[...] = x_ref[0, :]

    return pl.pallas_call(
        body,
        out_shape=jax.ShapeDtypeStruct((NWD,), jnp.float32),
    )(wt)


@functools.lru_cache(maxsize=None)
def _build(B):
    RPW = B // NW
    CH = 64
    CHN = CH * F
    NCHUNK = RPW // CH
    slices = []
    o = 0
    while o < CHN:
        n = min(GC, CHN - o)
        slices.append((o, n))
        o += n

    mesh = plsc.VectorSubcoreMesh(core_axis_name="c", subcore_axis_name="s")

    def _bufset():
        return [
            pltpu.VMEM((CHN + L,), jnp.int32),
            pltpu.VMEM((CHN,), jnp.int32),
            pltpu.VMEM((CHN + L,), jnp.float32),
            pltpu.VMEM((CHN, L), jnp.float32),
            pltpu.VMEM((CHN + L,), jnp.float32),
            pltpu.SemaphoreType.DMA,
            pltpu.SemaphoreType.DMA,
        ]

    @functools.partial(
        pl.kernel,
        out_type=jax.ShapeDtypeStruct((B,), jnp.float32),
        mesh=mesh,
        compiler_params=pltpu.CompilerParams(use_tc_tiling_on_sc=False),
        scratch_types=_bufset() + _bufset() + [
            pltpu.VMEM((CH,), jnp.float32),
            pltpu.VMEM((L,), jnp.float32),
        ],
    )
    def fm(idx_hbm, idx8_hbm, vals_hbm, w_hbm, Vb_hbm, b_hbm,
           y_hbm, *scr):
        bufs = (scr[0:7], scr[7:14])
        ybuf, bv = scr[14], scr[15]
        cid = lax.axis_index("c")
        sid = lax.axis_index("s")
        wid = sid * NC + cid
        base = wid * RPW
        pltpu.sync_copy(b_hbm, bv)
        iota = lax.iota(jnp.int32, L)
        m10 = iota < (F - L)
        fzero = jnp.zeros((L,), jnp.float32)
        bvec = bv[...]

        def issue(ci, bs):
            idxv, idx8v, vvals, vrows, wrows, sem_v, sem_w = bs
            rowbase = base + ci * CH
            pltpu.sync_copy(idx_hbm.at[pl.ds(rowbase * F, CHN)],
                            idxv.at[pl.ds(0, CHN)])
            pltpu.sync_copy(idx8_hbm.at[pl.ds(rowbase * F, CHN)], idx8v)
            pltpu.sync_copy(vals_hbm.at[pl.ds(rowbase * F, CHN)],
                            vvals.at[pl.ds(0, CHN)])
            cps = []
            for (so, sn) in slices:
                sl = pl.ds(so, sn)
                cps.append(pltpu.async_copy(
                    Vb_hbm.at[idx8v.at[sl]], vrows.at[sl, :], sem_v))
                cps.append(pltpu.async_copy(
                    w_hbm.at[idxv.at[sl]], wrows.at[sl], sem_w))
            return cps

        def compute(ci, bs):
            idxv, idx8v, vvals, vrows, wrows, sem_v, sem_w = bs
            rowbase = base + ci * CH

            def grp(g, c2):
                def row_body(rr, lvec):
                    r = g * L + rr
                    off = r * F
                    va = vvals[pl.ds(off, L)]
                    vb = vvals[pl.ds(off + L, L)]
                    accxv = fzero
                    accx2 = fzero
                    for f in range(F):
                        if f < L:
                            bf = _bcast_lane(va, f)
                        else:
                            bf = _bcast_lane(vb, f - L)
                        j = off + f
                        t = bf * vrows[j, :]
                        accxv = accxv + t
                        accx2 = accx2 + t * t
                    d = accxv * accxv - accx2
                    wa = wrows[pl.ds(off, L)]
                    wb = wrows[pl.ds(off + L, L)]
                    vbm = jnp.where(m10, vb, 0.0)
                    s = _lanesum(d, iota)
                    sumv = _lanesum(va + vbm, iota)
                    xw = _lanesum(va * wa + jnp.where(m10, vb * wb, 0.0),
                                  iota)
                    logit = xw + 0.5 * s / sumv
                    return jnp.where(iota == rr, logit, lvec)

                lvec = lax.fori_loop(0, L, row_body, fzero)
                y = 1.0 / (1.0 + jnp.exp(-(lvec + bvec)))
                ybuf[pl.ds(g * L, L)] = y
                return c2

            lax.fori_loop(0, CH // L, grp, 0)
            pltpu.sync_copy(ybuf, y_hbm.at[pl.ds(rowbase, CH)])

        pend = {0: issue(0, bufs[0])}
        for ci in range(NCHUNK):
            if ci + 1 < NCHUNK:
                pend[ci + 1] = issue(ci + 1, bufs[(ci + 1) & 1])
            for cp in pend.pop(ci):
                cp.wait()
            compute(ci, bufs[ci & 1])

    return fm


def kernel(indices, values, w, V, b):
    B = indices.shape[0]
    NB = 128000
    idx_flat = indices.reshape(-1).astype(jnp.int32)
    idxp_flat = (idx_flat % NB) * 8 + idx_flat // NB
    vals_flat = values.reshape(-1).astype(jnp.float32)
    w_flat = _tc_wflat(w.astype(jnp.float32).T)
    Vt = V.astype(jnp.float32).T
    NV = Vt.shape[1]
    pad7 = jnp.pad(Vt[:, 7 * NB:], ((0, 0), (0, NB * 8 - NV)))
    Vlin = _tc_relayout(Vt, pad7, NB).reshape(NB * 8, L)
    b16 = jnp.zeros((L,), jnp.float32) + b.reshape(-1)[0].astype(jnp.float32)
    return _build(B)(idx_flat, idxp_flat, vals_flat, w_flat,
                     Vlin, b16)
